# baseline (speedup 1.0000x reference)
import functools

import jax
import jax.numpy as jnp
from jax.experimental import pallas as pl
from jax.experimental.pallas import tpu as pltpu


def _density_body(x_ref, k_ref, o_ref, bc_ref, *, num_levels, d_chunk):
    bb, f = x_ref.shape
    c = pl.program_id(1)

    @pl.when(c == 0)
    def _prologue():
        xb = x_ref[...]
        levels = jnp.float32(num_levels - 1)
        idx = jnp.round(jnp.clip(xb, 0.0, 1.0) * levels)
        idx = jnp.clip(idx, 0.0, levels).astype(jnp.int16)
        for j in range(f):
            bc_ref[j] = jnp.broadcast_to(idx[:, j:j + 1], (bb, d_chunk))

    kw = k_ref[...].astype(jnp.bfloat16)
    sh = jnp.sum(kw, axis=0, keepdims=True) * jnp.bfloat16(0.5)
    base = (c * d_chunk).astype(jnp.int16)
    iota_c = jax.lax.broadcasted_iota(jnp.int16, (bb, d_chunk), 1) + base
    zero = jnp.zeros((), jnp.bfloat16)
    acc = jnp.where(iota_c < bc_ref[0], kw[0, :], zero)
    acc = acc - sh
    for j in range(1, f):
        acc = acc + jnp.where(iota_c < bc_ref[j], kw[j, :], zero)
    o_ref[...] = (acc + acc).astype(jnp.float32)


def kernel(x, key_weight, thermo_weight):
    batch, feats = x.shape
    d_dim = key_weight.shape[1]
    num_levels = thermo_weight.shape[0]
    block_b = 256
    d_chunk = 256
    grid = (batch // block_b, d_dim // d_chunk)
    return pl.pallas_call(
        functools.partial(_density_body, num_levels=num_levels,
                          d_chunk=d_chunk),
        grid=grid,
        in_specs=[
            pl.BlockSpec((block_b, feats), lambda i, j: (i, 0)),
            pl.BlockSpec((feats, d_chunk), lambda i, j: (0, j)),
        ],
        out_specs=pl.BlockSpec((block_b, d_chunk), lambda i, j: (i, j)),
        out_shape=jax.ShapeDtypeStruct((batch, d_dim), jnp.float32),
        scratch_shapes=[
            pltpu.VMEM((feats, block_b, d_chunk), jnp.int16),
        ],
    )(x, key_weight)


# speedup vs baseline: 1.6358x; 1.6358x over previous
import functools

import jax
import jax.numpy as jnp
from jax.experimental import pallas as pl
from jax.experimental.pallas import tpu as pltpu


def _density_body(x_ref, k_ref, o_ref, *, num_levels, d_chunk):
    xb = x_ref[...]
    bb, f = xb.shape
    d_dim = o_ref.shape[1]
    levels = jnp.float32(num_levels - 1)
    idx = jnp.round(jnp.clip(xb, 0.0, 1.0) * levels)
    idx = jnp.clip(idx, 0.0, levels).astype(jnp.int16)
    cols = [jnp.broadcast_to(idx[:, j:j + 1], (bb, d_chunk))
            for j in range(f)]
    d_iota = jax.lax.broadcasted_iota(jnp.int16, (bb, d_chunk), 1)
    kw_all = k_ref[...].astype(jnp.bfloat16)
    s_half = jnp.sum(kw_all, axis=0, keepdims=True) * jnp.bfloat16(0.5)
    zero = jnp.zeros((), jnp.bfloat16)
    for c in range(d_dim // d_chunk):
        iota_c = d_iota + jnp.int16(c * d_chunk)
        kw = kw_all[:, c * d_chunk:(c + 1) * d_chunk]
        acc = jnp.where(iota_c < cols[0], kw[0, :], zero)
        acc = acc - s_half[:, c * d_chunk:(c + 1) * d_chunk]
        for j in range(1, f):
            acc = acc + jnp.where(iota_c < cols[j], kw[j, :], zero)
        o_ref[:, c * d_chunk:(c + 1) * d_chunk] = (
            (acc + acc).astype(jnp.float32))


def kernel(x, key_weight, thermo_weight):
    batch, feats = x.shape
    d_dim = key_weight.shape[1]
    num_levels = thermo_weight.shape[0]
    block_b = 256
    grid = (batch // block_b,)
    return pl.pallas_call(
        functools.partial(_density_body, num_levels=num_levels,
                          d_chunk=256),
        grid=grid,
        compiler_params=pltpu.CompilerParams(
            dimension_semantics=("parallel",)),
        in_specs=[
            pl.BlockSpec((block_b, feats), lambda i: (i, 0)),
            pl.BlockSpec((feats, d_dim), lambda i: (0, 0)),
        ],
        out_specs=pl.BlockSpec((block_b, d_dim), lambda i: (i, 0)),
        out_shape=jax.ShapeDtypeStruct((batch, d_dim), jnp.float32),
    )(x, key_weight)
